# Initial kernel scaffold; baseline (speedup 1.0000x reference)
import functools

import jax
import jax.numpy as jnp
from jax import lax
from jax.experimental import pallas as pl
from jax.experimental.pallas import tpu as pltpu
from jax.experimental.pallas import tpu_sc as plsc

B = 16384
D = 32
N_NEG = 20
L = 16
NC = 2
NS = 16
NW = NC * NS
BPW = B // NW


def _sc_body(x_hbm, y_hbm, negt_hbm, wh_hbm, wo_hbm, pos_out, neg_out,
             xi, yi, ni, h, t, a, pos_v, neg_v, sem):
  wid = lax.axis_index("s") * NC + lax.axis_index("c")
  base = wid * BPW

  pltpu.sync_copy(x_hbm.at[pl.ds(base, BPW)], xi)
  pltpu.sync_copy(y_hbm.at[pl.ds(base, BPW)], yi)
  pltpu.sync_copy(negt_hbm.at[:, pl.ds(base, BPW)], ni)

  cp_h = pltpu.async_copy(wh_hbm.at[xi], h, sem)
  cp_t = pltpu.async_copy(wo_hbm.at[yi], t, sem)
  cp_a = pltpu.async_copy(wo_hbm.at[ni.at[0]], a, sem)
  cp_h.wait()
  cp_t.wait()
  cp_a.wait()
  for n in range(1, N_NEG):
    pltpu.async_copy(wo_hbm.at[ni.at[n]], a, sem, add=True).wait()

  def row(b, _):
    h0 = h[b, pl.ds(0, L)]
    h1 = h[b, pl.ds(L, L)]
    pos_v[b] = jnp.sum(t[b, pl.ds(0, L)] * h0 + t[b, pl.ds(L, L)] * h1)
    neg_v[b] = jnp.sum(a[b, pl.ds(0, L)] * h0 + a[b, pl.ds(L, L)] * h1)
    return 0

  lax.fori_loop(0, BPW, row, 0)

  pltpu.sync_copy(pos_v, pos_out.at[pl.ds(base, BPW)])
  pltpu.sync_copy(neg_v, neg_out.at[pl.ds(base, BPW)])


@jax.jit
def _sc_scores(x, y, neg_t, w_hidden, w_output):
  mesh = plsc.VectorSubcoreMesh(core_axis_name="c", subcore_axis_name="s")
  return pl.kernel(
      _sc_body,
      out_type=(
          jax.ShapeDtypeStruct((B,), jnp.float32),
          jax.ShapeDtypeStruct((B,), jnp.float32),
      ),
      mesh=mesh,
      scratch_types=[
          pltpu.VMEM((BPW,), jnp.int32),
          pltpu.VMEM((BPW,), jnp.int32),
          pltpu.VMEM((N_NEG, BPW), jnp.int32),
          pltpu.VMEM((BPW, D), jnp.float32),
          pltpu.VMEM((BPW, D), jnp.float32),
          pltpu.VMEM((BPW, D), jnp.float32),
          pltpu.VMEM((BPW,), jnp.float32),
          pltpu.VMEM((BPW,), jnp.float32),
          pltpu.SemaphoreType.DMA,
      ],
  )(x, y, neg_t, w_hidden, w_output)


def _log_sigmoid(z):
  return jnp.minimum(z, 0.0) - jnp.log1p(jnp.exp(-jnp.abs(z)))


def _loss_body(pos_ref, neg_ref, out_ref):
  pos = pos_ref[...]
  neg = -neg_ref[...]
  loss = _log_sigmoid(pos) + _log_sigmoid(neg)
  out_ref[0, 0] = -jnp.sum(loss) / B


@jax.jit
def _tc_loss(pos, neg):
  out = pl.pallas_call(
      _loss_body,
      out_shape=jax.ShapeDtypeStruct((1, 1), jnp.float32),
      out_specs=pl.BlockSpec(memory_space=pltpu.SMEM),
  )(pos.reshape(128, 128), neg.reshape(128, 128))
  return out[0, 0]


def kernel(x, y, negative_batch, W_hidden, W_output):
  xf = x.reshape(B)
  yf = y.reshape(B)
  neg_t = negative_batch.T
  pos, negdot = _sc_scores(xf, yf, neg_t, W_hidden, W_output)
  return _tc_loss(pos, negdot)


# speedup vs baseline: 1.0680x; 1.0680x over previous
import functools

import jax
import jax.numpy as jnp
from jax import lax
from jax.experimental import pallas as pl
from jax.experimental.pallas import tpu as pltpu
from jax.experimental.pallas import tpu_sc as plsc

B = 16384
D = 32
N_NEG = 20
L = 16
NC = 2
NS = 16
NW = NC * NS
BPW = B // NW


def _sc_body(x_hbm, y_hbm, negt_hbm, wh_hbm, wo_hbm, pos_out, neg_out,
             xi, yi, ni, h, t, a, pos_v, neg_v, sem):
  wid = lax.axis_index("s") * NC + lax.axis_index("c")
  base = wid * BPW

  pltpu.sync_copy(x_hbm.at[pl.ds(base, BPW)], xi)
  pltpu.sync_copy(y_hbm.at[pl.ds(base, BPW)], yi)
  pltpu.sync_copy(negt_hbm.at[:, pl.ds(base, BPW)], ni)

  cp_h = pltpu.async_copy(wh_hbm.at[xi], h, sem)
  cp_t = pltpu.async_copy(wo_hbm.at[yi], t, sem)
  cp_a = pltpu.async_copy(wo_hbm.at[ni.at[0]], a, sem)
  cp_h.wait()
  cp_t.wait()
  cp_a.wait()
  for n in range(1, N_NEG):
    pltpu.async_copy(wo_hbm.at[ni.at[n]], a, sem, add=True).wait()

  def row16(i, _):
    b = i * L
    rows = b + lax.iota(jnp.int32, L)
    pacc = jnp.zeros((L,), jnp.float32)
    nacc = jnp.zeros((L,), jnp.float32)
    for d in range(D):
      cols = jnp.full((L,), d, jnp.int32)
      hv = plsc.load_gather(h, [rows, cols])
      pacc = pacc + plsc.load_gather(t, [rows, cols]) * hv
      nacc = nacc + plsc.load_gather(a, [rows, cols]) * hv
    pos_v[pl.ds(b, L)] = pacc
    neg_v[pl.ds(b, L)] = nacc
    return 0

  lax.fori_loop(0, BPW // L, row16, 0)

  pltpu.sync_copy(pos_v, pos_out.at[pl.ds(base, BPW)])
  pltpu.sync_copy(neg_v, neg_out.at[pl.ds(base, BPW)])


@jax.jit
def _sc_scores(x, y, neg_t, w_hidden, w_output):
  mesh = plsc.VectorSubcoreMesh(core_axis_name="c", subcore_axis_name="s")
  return pl.kernel(
      _sc_body,
      out_type=(
          jax.ShapeDtypeStruct((B,), jnp.float32),
          jax.ShapeDtypeStruct((B,), jnp.float32),
      ),
      mesh=mesh,
      compiler_params=pltpu.CompilerParams(
          needs_layout_passes=False, use_tc_tiling_on_sc=False),
      scratch_types=[
          pltpu.VMEM((BPW,), jnp.int32),
          pltpu.VMEM((BPW,), jnp.int32),
          pltpu.VMEM((N_NEG, BPW), jnp.int32),
          pltpu.VMEM((BPW, D), jnp.float32),
          pltpu.VMEM((BPW, D), jnp.float32),
          pltpu.VMEM((BPW, D), jnp.float32),
          pltpu.VMEM((BPW,), jnp.float32),
          pltpu.VMEM((BPW,), jnp.float32),
          pltpu.SemaphoreType.DMA,
      ],
  )(x, y, neg_t, w_hidden, w_output)


def _log_sigmoid(z):
  return jnp.minimum(z, 0.0) - jnp.log1p(jnp.exp(-jnp.abs(z)))


def _loss_body(pos_ref, neg_ref, out_ref):
  pos = pos_ref[...]
  neg = -neg_ref[...]
  loss = _log_sigmoid(pos) + _log_sigmoid(neg)
  out_ref[0, 0] = -jnp.sum(loss) / B


@jax.jit
def _tc_loss(pos, neg):
  out = pl.pallas_call(
      _loss_body,
      out_shape=jax.ShapeDtypeStruct((1, 1), jnp.float32),
      out_specs=pl.BlockSpec(memory_space=pltpu.SMEM),
  )(pos.reshape(128, 128), neg.reshape(128, 128))
  return out[0, 0]


def kernel(x, y, negative_batch, W_hidden, W_output):
  xf = x.reshape(B)
  yf = y.reshape(B)
  neg_t = negative_batch.T
  pos, negdot = _sc_scores(xf, yf, neg_t, W_hidden, W_output)
  return _tc_loss(pos, negdot)
